# baseline (speedup 1.0000x reference)
import jax
import jax.numpy as jnp
from jax.experimental import pallas as pl
from jax.experimental.pallas import tpu as pltpu
from jax.experimental.pallas import tpu_sc as plsc

N_FLAT = 18432
NUM_TOKENS = 8192
TOKEN_DIM = 256

BN = 1024
CK = 8192
GW = 128


def _argmin_body(xsq_ref, esq_ref, x2_ref, e_ref, idx_ref):
    x2 = x2_ref[...]
    xsq = xsq_ref[...]
    nchunk = NUM_TOKENS // CK
    iota = jax.lax.broadcasted_iota(
        jnp.int32, (BN, CK), 1).astype(jnp.float32)

    def step(c, carry):
        bmin, bidx = carry
        e_c = e_ref[pl.ds(c * CK, CK), :]
        esq_c = esq_ref[:, pl.ds(c * CK, CK)]
        mm = jax.lax.dot_general(
            x2, e_c, (((1,), (1,)), ((), ())),
            preferred_element_type=jnp.float32)
        d = (xsq + esq_c) + mm
        cmin = jnp.min(d, axis=1, keepdims=True)
        cidx = jnp.min(jnp.where(d == cmin, iota, float(CK)),
                       axis=1, keepdims=True) + float(CK) * c
        take = cmin < bmin
        return (jnp.where(take, cmin, bmin), jnp.where(take, cidx, bidx))

    init = (jnp.full((BN, 1), jnp.inf, jnp.float32),
            jnp.zeros((BN, 1), jnp.float32))
    _, bidx = jax.lax.fori_loop(0, nchunk, step, init)
    idx_ref[...] = bidx.astype(jnp.int32)


def _tc_argmin(xsq, esq, x, e):
    n = x.shape[0]
    return pl.pallas_call(
        _argmin_body,
        grid=(n // BN,),
        in_specs=[
            pl.BlockSpec((BN, 1), lambda n: (n, 0)),
            pl.BlockSpec((1, NUM_TOKENS), lambda n: (0, 0)),
            pl.BlockSpec((BN, TOKEN_DIM), lambda n: (n, 0)),
            pl.BlockSpec((NUM_TOKENS, TOKEN_DIM), lambda n: (0, 0)),
        ],
        out_specs=pl.BlockSpec((BN, 1), lambda n: (n, 0)),
        out_shape=jax.ShapeDtypeStruct((n, 1), jnp.int32),
        compiler_params=pltpu.CompilerParams(
            dimension_semantics=("parallel",)),
    )(xsq, esq, x, e)


def _sc_gather(e, idx_row):
    n = idx_row.shape[1]

    @pl.kernel(
        out_type=jax.ShapeDtypeStruct((n, TOKEN_DIM), jnp.float32),
        mesh=plsc.VectorSubcoreMesh(core_axis_name="core",
                                    subcore_axis_name="subcore"))
    def gk(e_hbm, i_hbm, o_hbm):
        def body(i_vmem, o_vmem):
            pltpu.sync_copy(e_hbm.at[i_vmem.at[0]], o_vmem)

        pltpu.emit_pipeline(
            body,
            grid=(n // GW,),
            in_specs=[pl.BlockSpec((1, GW), index_map=lambda i: (0, i))],
            out_specs=[pl.BlockSpec((GW, TOKEN_DIM),
                                    index_map=lambda i: (i, 0))],
            core_axis_name=("core", "subcore"),
            dimension_semantics=(pltpu.PARALLEL,),
        )(i_hbm, o_hbm)

    return gk(e, idx_row)


NSPLIT = 2


@jax.jit
def kernel(inputs_flatten, embed):
    xsq = jnp.sum(inputs_flatten ** 2, axis=1, keepdims=True)
    esq = jnp.sum(embed ** 2, axis=1)[None, :]
    x2 = -2.0 * inputs_flatten
    h = N_FLAT // NSPLIT
    idxs, qs = [], []
    for k in range(NSPLIT):
        sl = slice(k * h, (k + 1) * h)
        idx_k = _tc_argmin(xsq[sl], esq, x2[sl], embed)
        qs.append(_sc_gather(embed, idx_k.reshape(1, h)))
        idxs.append(idx_k)
    return (jnp.concatenate(qs, axis=0), jnp.concatenate(idxs, axis=0))


# speedup vs baseline: 1.0662x; 1.0662x over previous
import jax
import jax.numpy as jnp
from jax.experimental import pallas as pl
from jax.experimental.pallas import tpu as pltpu
from jax.experimental.pallas import tpu_sc as plsc

N_FLAT = 18432
NUM_TOKENS = 8192
TOKEN_DIM = 256

BN = 1024
CK = 8192
GW = 128


def _argmin_body(xsq_ref, esq_ref, x2_ref, e_ref, idx_ref):
    x2 = x2_ref[...]
    xsq = xsq_ref[...]
    nchunk = NUM_TOKENS // CK
    iota = jax.lax.broadcasted_iota(
        jnp.int32, (BN, CK), 1).astype(jnp.float32)

    def step(c, carry):
        bmin, bidx = carry
        e_c = e_ref[pl.ds(c * CK, CK), :]
        esq_c = esq_ref[:, pl.ds(c * CK, CK)]
        mm = jax.lax.dot_general(
            x2, e_c, (((1,), (1,)), ((), ())),
            preferred_element_type=jnp.float32)
        d = (xsq + esq_c) + mm
        cmin = jnp.min(d, axis=1, keepdims=True)
        cidx = jnp.min(jnp.where(d == cmin, iota, float(CK)),
                       axis=1, keepdims=True) + float(CK) * c
        take = cmin < bmin
        return (jnp.where(take, cmin, bmin), jnp.where(take, cidx, bidx))

    init = (jnp.full((BN, 1), jnp.inf, jnp.float32),
            jnp.zeros((BN, 1), jnp.float32))
    _, bidx = jax.lax.fori_loop(0, nchunk, step, init)
    idx_ref[...] = bidx.astype(jnp.int32)


def _tc_argmin(xsq, esq, x, e):
    n = x.shape[0]
    return pl.pallas_call(
        _argmin_body,
        grid=(n // BN,),
        in_specs=[
            pl.BlockSpec((BN, 1), lambda n: (n, 0)),
            pl.BlockSpec((1, NUM_TOKENS), lambda n: (0, 0)),
            pl.BlockSpec((BN, TOKEN_DIM), lambda n: (n, 0)),
            pl.BlockSpec((NUM_TOKENS, TOKEN_DIM), lambda n: (0, 0)),
        ],
        out_specs=pl.BlockSpec((BN, 1), lambda n: (n, 0)),
        out_shape=jax.ShapeDtypeStruct((n, 1), jnp.int32),
        compiler_params=pltpu.CompilerParams(
            dimension_semantics=("parallel",)),
    )(xsq, esq, x, e)


def _sc_gather(e, idx_row):
    n = idx_row.shape[1]

    @pl.kernel(
        out_type=jax.ShapeDtypeStruct((n, TOKEN_DIM), jnp.float32),
        mesh=plsc.VectorSubcoreMesh(core_axis_name="core",
                                    subcore_axis_name="subcore"))
    def gk(e_hbm, i_hbm, o_hbm):
        def body(i_vmem, o_vmem):
            pltpu.sync_copy(e_hbm.at[i_vmem.at[0]], o_vmem)

        pltpu.emit_pipeline(
            body,
            grid=(n // GW,),
            in_specs=[pl.BlockSpec((1, GW), index_map=lambda i: (0, i))],
            out_specs=[pl.BlockSpec((GW, TOKEN_DIM),
                                    index_map=lambda i: (i, 0))],
            core_axis_name=("core", "subcore"),
            dimension_semantics=(pltpu.PARALLEL,),
        )(i_hbm, o_hbm)

    return gk(e, idx_row)


@jax.jit
def kernel(inputs_flatten, embed):
    xsq = jnp.sum(inputs_flatten ** 2, axis=1, keepdims=True)
    esq = jnp.sum(embed ** 2, axis=1)[None, :]
    x2 = -2.0 * inputs_flatten
    idx = _tc_argmin(xsq, esq, x2, embed)
    quantize = _sc_gather(embed, idx.reshape(1, N_FLAT))
    return (quantize, idx)


# speedup vs baseline: 1.0881x; 1.0205x over previous
import jax
import jax.numpy as jnp
from jax.experimental import pallas as pl
from jax.experimental.pallas import tpu as pltpu
from jax.experimental.pallas import tpu_sc as plsc

N_FLAT = 18432
NUM_TOKENS = 8192
TOKEN_DIM = 256

BN = 1024
CK = 8192
GW = 128


def _argmin_body(xsq_ref, esq_ref, x_ref, e_ref, idx_ref, row_ref):
    x2 = x_ref[...] * -2.0
    xsq = xsq_ref[...]
    iota = jax.lax.broadcasted_iota(
        jnp.int32, (BN, CK), 1).astype(jnp.float32)

    mm = jax.lax.dot_general(
        x2, e_ref[...], (((1,), (1,)), ((), ())),
        preferred_element_type=jnp.float32)
    d = (xsq + esq_ref[...]) + mm
    cmin = jnp.min(d, axis=1, keepdims=True)
    bidx = jnp.min(jnp.where(d == cmin, iota, float(CK)),
                   axis=1, keepdims=True)
    bidx = bidx.astype(jnp.int32)
    idx_ref[...] = bidx
    row_ref[...] = bidx.reshape(1, BN)


def _tc_argmin(xsq, esq, x, e):
    n = x.shape[0]
    return pl.pallas_call(
        _argmin_body,
        grid=(n // BN,),
        in_specs=[
            pl.BlockSpec((BN, 1), lambda n: (n, 0)),
            pl.BlockSpec((1, NUM_TOKENS), lambda n: (0, 0)),
            pl.BlockSpec((BN, TOKEN_DIM), lambda n: (n, 0)),
            pl.BlockSpec((NUM_TOKENS, TOKEN_DIM), lambda n: (0, 0)),
        ],
        out_specs=[
            pl.BlockSpec((BN, 1), lambda n: (n, 0)),
            pl.BlockSpec((1, BN), lambda n: (0, n)),
        ],
        out_shape=[
            jax.ShapeDtypeStruct((n, 1), jnp.int32),
            jax.ShapeDtypeStruct((1, n), jnp.int32),
        ],
        compiler_params=pltpu.CompilerParams(
            dimension_semantics=("parallel",)),
    )(xsq, esq, x, e)


def _sc_gather(e, idx_row):
    n = idx_row.shape[1]

    @pl.kernel(
        out_type=jax.ShapeDtypeStruct((n, TOKEN_DIM), jnp.float32),
        mesh=plsc.VectorSubcoreMesh(core_axis_name="core",
                                    subcore_axis_name="subcore"))
    def gk(e_hbm, i_hbm, o_hbm):
        def body(i_vmem, o_vmem):
            pltpu.sync_copy(e_hbm.at[i_vmem.at[0]], o_vmem)

        pltpu.emit_pipeline(
            body,
            grid=(n // GW,),
            in_specs=[pl.BlockSpec((1, GW), index_map=lambda i: (0, i))],
            out_specs=[pl.BlockSpec((GW, TOKEN_DIM),
                                    index_map=lambda i: (i, 0))],
            core_axis_name=("core", "subcore"),
            dimension_semantics=(pltpu.PARALLEL,),
        )(i_hbm, o_hbm)

    return gk(e, idx_row)


@jax.jit
def kernel(inputs_flatten, embed):
    xsq = jnp.sum(inputs_flatten ** 2, axis=1, keepdims=True)
    esq = jnp.sum(embed ** 2, axis=1)[None, :]
    idx, idx_row = _tc_argmin(xsq, esq, inputs_flatten, embed)
    quantize = _sc_gather(embed, idx_row)
    return (quantize, idx)
